# baseline (speedup 1.0000x reference)
import functools

import jax
import jax.numpy as jnp
from jax import lax
from jax.experimental import pallas as pl
from jax.experimental.pallas import tpu as pltpu
from jax.experimental.pallas import tpu_sc as plsc

N_EMBD = 768
NUM_EXPERTS = 64
TOP_K = 8
TOKENS = 32768
BLOCK = 4096
CHUNKS = 4
CHUNK = TOKENS // CHUNKS

NC, NS, L = 2, 16, 16
NW = NC * NS
TOK_PER_W = CHUNK // NW
NGROUP = TOK_PER_W // L

_IMASK = NUM_EXPERTS - 1


def _keys_body(x_ref, w_ref, keys_ref):
    logits_t = jax.lax.dot_general(
        w_ref[...], x_ref[...],
        dimension_numbers=(((1,), (1,)), ((), ())),
        preferred_element_type=jnp.float32,
    )
    si = jax.lax.bitcast_convert_type(logits_t, jnp.int32)
    sortable = si ^ (jax.lax.shift_right_arithmetic(si, 31) & 0x7FFFFFFF)
    rev_iota = _IMASK - jax.lax.broadcasted_iota(
        jnp.int32, (NUM_EXPERTS, BLOCK), 0)
    keys_ref[...] = (sortable & ~_IMASK) | rev_iota


def _tc_keys_chunk(x, w_gate, c):
    return pl.pallas_call(
        _keys_body,
        grid=(CHUNK // BLOCK,),
        in_specs=[
            pl.BlockSpec((BLOCK, N_EMBD),
                         lambda i, c=c: (c * (CHUNK // BLOCK) + i, 0)),
            pl.BlockSpec((NUM_EXPERTS, N_EMBD), lambda i: (0, 0)),
        ],
        out_specs=pl.BlockSpec((NUM_EXPERTS, BLOCK), lambda i: (0, i)),
        out_shape=jax.ShapeDtypeStruct((NUM_EXPERTS, CHUNK), jnp.int32),
    )(x, w_gate)


_SC_MESH = plsc.VectorSubcoreMesh(
    core_axis_name="c", subcore_axis_name="s", num_cores=NC, num_subcores=NS)


@functools.partial(
    pl.kernel,
    out_type=(
        jax.ShapeDtypeStruct((TOP_K, CHUNK), jnp.int32),
        jax.ShapeDtypeStruct((TOP_K, CHUNK), jnp.float32),
    ),
    mesh=_SC_MESH,
    scratch_types=[
        pltpu.VMEM((NUM_EXPERTS, TOK_PER_W), jnp.int32),
        pltpu.VMEM((TOP_K, TOK_PER_W), jnp.int32),
        pltpu.VMEM((TOP_K, TOK_PER_W), jnp.float32),
    ],
)
def _sc_topk(keys_hbm, idx_hbm, score_hbm, keys_v, idx_v, score_v):
    wid = lax.axis_index("s") * NC + lax.axis_index("c")
    base = wid * TOK_PER_W
    pltpu.sync_copy(keys_hbm.at[:, pl.ds(base, TOK_PER_W)], keys_v)

    def group(g, carry):
        off = g * L
        neg = jnp.full((L,), -(2 ** 31), jnp.int32)
        best = [neg] * TOP_K
        for e in range(NUM_EXPERTS):
            v = keys_v[e, pl.ds(off, L)]
            for j in range(TOP_K):
                hi = jnp.maximum(best[j], v)
                v = jnp.minimum(best[j], v)
                best[j] = hi
        vals = []
        for j in range(TOP_K):
            k = best[j]
            idx_v[j, pl.ds(off, L)] = _IMASK - (k & _IMASK)
            vs = k & ~_IMASK
            vsi = vs ^ (lax.shift_right_arithmetic(vs, 31) & 0x7FFFFFFF)
            vals.append(lax.bitcast_convert_type(vsi, jnp.float32))
        exps = [jnp.exp(v - vals[0]) for v in vals]
        tot = exps[0]
        for j in range(1, TOP_K):
            tot = tot + exps[j]
        for j in range(TOP_K):
            score_v[j, pl.ds(off, L)] = exps[j] / tot
        return carry

    lax.fori_loop(0, NGROUP, group, 0)
    pltpu.sync_copy(idx_v, idx_hbm.at[:, pl.ds(base, TOK_PER_W)])
    pltpu.sync_copy(score_v, score_hbm.at[:, pl.ds(base, TOK_PER_W)])


@jax.jit
def kernel(x, w_gate):
    idx_parts, score_parts = [], []
    for c in range(CHUNKS):
        keys_c = _tc_keys_chunk(x, w_gate, c)
        idx_c, score_c = _sc_topk(keys_c)
        idx_parts.append(idx_c)
        score_parts.append(score_c)
    return (jnp.concatenate(idx_parts, axis=1).T,
            jnp.concatenate(score_parts, axis=1).T)


# speedup vs baseline: 1.0662x; 1.0662x over previous
import functools

import jax
import jax.numpy as jnp
from jax import lax
from jax.experimental import pallas as pl
from jax.experimental.pallas import tpu as pltpu
from jax.experimental.pallas import tpu_sc as plsc

N_EMBD = 768
NUM_EXPERTS = 64
TOP_K = 8
TOKENS = 32768
BLOCK = 4096

NC, NS, L = 2, 16, 16
NW = NC * NS
TOK_PER_W = TOKENS // NW
SLAB = 256
NSLAB = TOK_PER_W // SLAB
NGROUP = SLAB // L

_IMASK = NUM_EXPERTS - 1

_NET8 = ((0, 1), (2, 3), (4, 5), (6, 7), (0, 2), (1, 3), (4, 6), (5, 7),
         (1, 2), (5, 6), (0, 4), (3, 7), (1, 5), (2, 6), (1, 4), (3, 6),
         (2, 4), (3, 5), (3, 4))


def _keys_body(x_ref, w_ref, keys_ref):
    logits_t = jax.lax.dot_general(
        w_ref[...], x_ref[...],
        dimension_numbers=(((1,), (1,)), ((), ())),
        preferred_element_type=jnp.float32,
    )
    si = jax.lax.bitcast_convert_type(logits_t, jnp.int32)
    sortable = si ^ (jax.lax.shift_right_arithmetic(si, 31) & 0x7FFFFFFF)
    rev_iota = _IMASK - jax.lax.broadcasted_iota(
        jnp.int32, (NUM_EXPERTS, BLOCK), 0)
    keys_ref[...] = (sortable & ~_IMASK) | rev_iota


def _tc_keys(x, w_gate):
    return pl.pallas_call(
        _keys_body,
        grid=(TOKENS // BLOCK,),
        in_specs=[
            pl.BlockSpec((BLOCK, N_EMBD), lambda i: (i, 0)),
            pl.BlockSpec((NUM_EXPERTS, N_EMBD), lambda i: (0, 0)),
        ],
        out_specs=pl.BlockSpec((NUM_EXPERTS, BLOCK), lambda i: (0, i)),
        out_shape=jax.ShapeDtypeStruct((NUM_EXPERTS, TOKENS), jnp.int32),
    )(x, w_gate)


_SC_MESH = plsc.VectorSubcoreMesh(
    core_axis_name="c", subcore_axis_name="s", num_cores=NC, num_subcores=NS)


@functools.partial(
    pl.kernel,
    out_type=(
        jax.ShapeDtypeStruct((TOP_K, TOKENS), jnp.int32),
        jax.ShapeDtypeStruct((TOP_K, TOKENS), jnp.float32),
    ),
    mesh=_SC_MESH,
    scratch_types=[
        pltpu.VMEM((2, NUM_EXPERTS, SLAB), jnp.int32),
        pltpu.VMEM((TOP_K, TOK_PER_W), jnp.int32),
        pltpu.VMEM((TOP_K, TOK_PER_W), jnp.float32),
        pltpu.SemaphoreType.DMA,
        pltpu.SemaphoreType.DMA,
    ],
)
def _sc_topk(keys_hbm, idx_hbm, score_hbm, keys_v, idx_v, score_v,
             sem0, sem1):
    wid = lax.axis_index("s") * NC + lax.axis_index("c")
    base = wid * TOK_PER_W
    sems = (sem0, sem1)

    def start_slab(s):
        return pltpu.async_copy(
            keys_hbm.at[:, pl.ds(base + s * SLAB, SLAB)],
            keys_v.at[s % 2], sems[s % 2])

    copies = {0: start_slab(0)}
    for s in range(NSLAB):
        copies[s].wait()
        if s + 1 < NSLAB:
            copies[s + 1] = start_slab(s + 1)
        buf = s % 2

        def group(g, carry, buf=buf, s=s):
            off = g * L
            out_off = s * SLAB + off
            best = [keys_v[buf, e, pl.ds(off, L)] for e in range(TOP_K)]
            for (a, b) in _NET8:
                hi = jnp.maximum(best[a], best[b])
                best[b] = jnp.minimum(best[a], best[b])
                best[a] = hi
            for e in range(TOP_K, NUM_EXPERTS):
                v = keys_v[buf, e, pl.ds(off, L)]
                for j in range(TOP_K):
                    hi = jnp.maximum(best[j], v)
                    v = jnp.minimum(best[j], v)
                    best[j] = hi
            vals = []
            for j in range(TOP_K):
                k = best[j]
                idx_v[j, pl.ds(out_off, L)] = _IMASK - (k & _IMASK)
                vs = k & ~_IMASK
                vsi = vs ^ (lax.shift_right_arithmetic(vs, 31) & 0x7FFFFFFF)
                vals.append(lax.bitcast_convert_type(vsi, jnp.float32))
            exps = [jnp.exp(v - vals[0]) for v in vals]
            tot = exps[0]
            for j in range(1, TOP_K):
                tot = tot + exps[j]
            for j in range(TOP_K):
                score_v[j, pl.ds(out_off, L)] = exps[j] / tot
            return carry

        lax.fori_loop(0, NGROUP, group, 0)

    pltpu.sync_copy(idx_v, idx_hbm.at[:, pl.ds(base, TOK_PER_W)])
    pltpu.sync_copy(score_v, score_hbm.at[:, pl.ds(base, TOK_PER_W)])


@jax.jit
def kernel(x, w_gate):
    keys = _tc_keys(x, w_gate)
    idx_t, score_t = _sc_topk(keys)
    return idx_t.T, score_t.T
